# baseline (speedup 1.0000x reference)
import jax
import jax.numpy as jnp
from jax import lax
from jax.experimental import pallas as pl
from jax.experimental.pallas import tpu as pltpu
from jax.experimental.pallas import tpu_sc as plsc

D = 128
S = 10
RPN = 12
NC, NS, L = 2, 16, 16
NW = NC * NS
CHUNK = 16
CR = CHUNK * RPN
IPG = 96
NG = CR // IPG
FMT = plsc.PackFormat.INTERLEAVED
DW = D // 2


def _sc_body(idx_hbm, table_hbm, out_hbm,
             idx_v, rows0, rows1, outc0, outc1,
             gsem0, gsem1, osem0, osem1, *, bpw, nchunks):
    wid = lax.axis_index("s") * NC + lax.axis_index("c")
    base = wid * bpw
    rows = (rows0, rows1)
    outc = (outc0, outc1)
    gsem = (gsem0, gsem1)
    osem = (osem0, osem1)

    pltpu.sync_copy(idx_hbm.at[pl.ds(base * RPN, bpw * RPN)], idx_v)

    def fire_gathers(b, c):
        for j in range(NG):
            pltpu.async_copy(
                table_hbm.at[idx_v.at[pl.ds(c * CR + j * IPG, IPG)]],
                rows[b].at[pl.ds(j * IPG, IPG)], gsem[b])

    def drain_gathers(b):
        for j in range(NG):
            pltpu.make_async_copy(
                table_hbm.at[idx_v.at[pl.ds(j * IPG, IPG)]],
                rows[b].at[pl.ds(j * IPG, IPG)], gsem[b]).wait()

    def compute(b):
        MASK = jnp.int32(-65536)
        for i in range(CHUNK):
            r0 = i * RPN
            for v in range(DW // L):
                sl = pl.ds(v * L, L)

                def halves(r):
                    w = rows[b][r, sl]
                    lo = lax.bitcast_convert_type(w << 16, jnp.float32)
                    hi = lax.bitcast_convert_type(w & MASK, jnp.float32)
                    return lo, hi

                acc0, acc1 = halves(r0)
                for s in range(1, S + 1):
                    x0, x1 = halves(r0 + s)
                    acc0 = acc0 + x0
                    acc1 = acc1 + x1
                outc[b][i, pl.ds(v * L, L)] = jnp.maximum(acc0, 0.0)
                outc[b][i, pl.ds(DW + v * L, L)] = jnp.maximum(acc1, 0.0)

    def fire_out(b, c):
        pltpu.async_copy(outc[b], out_hbm.at[pl.ds(base + c * CHUNK, CHUNK)],
                         osem[b])

    def drain_out(b):
        pltpu.make_async_copy(outc[b], out_hbm.at[pl.ds(base, CHUNK)],
                              osem[b]).wait()

    fire_gathers(0, 0)

    def superstep(ss, carry):
        c0 = 2 * ss

        @pl.when(ss > 0)
        def _():
            drain_out(1)
        fire_gathers(1, c0 + 1)
        drain_gathers(0)

        @pl.when(ss > 0)
        def _():
            drain_out(0)
        compute(0)
        fire_out(0, c0)

        @pl.when(c0 + 2 < nchunks)
        def _():
            fire_gathers(0, c0 + 2)
        drain_gathers(1)
        compute(1)
        fire_out(1, c0 + 1)
        return carry

    lax.fori_loop(0, nchunks // 2, superstep, 0)
    drain_out(0)
    drain_out(1)


def _make_sc(Bp):
    bpw = Bp // NW
    nchunks = bpw // CHUNK
    assert nchunks % 2 == 0
    mesh = plsc.VectorSubcoreMesh(core_axis_name="c", subcore_axis_name="s")

    def body(idx_hbm, table_hbm, out_hbm, *scratch):
        _sc_body(idx_hbm, table_hbm, out_hbm, *scratch,
                 bpw=bpw, nchunks=nchunks)

    return pl.kernel(
        body,
        out_type=jax.ShapeDtypeStruct((Bp, D), jnp.float32),
        mesh=mesh,
        compiler_params=pltpu.CompilerParams(use_tc_tiling_on_sc=False),
        scratch_types=[
            pltpu.VMEM((bpw * RPN,), jnp.int32),
            pltpu.VMEM((CR, DW), jnp.int32),
            pltpu.VMEM((CR, DW), jnp.int32),
            pltpu.VMEM((CHUNK, D), jnp.float32),
            pltpu.VMEM((CHUNK, D), jnp.float32),
            pltpu.SemaphoreType.DMA,
            pltpu.SemaphoreType.DMA,
            pltpu.SemaphoreType.DMA,
            pltpu.SemaphoreType.DMA,
        ],
    )


def _table_body(w_ref, f_ref, out_ref):
    f = f_ref[...]
    w = w_ref[0]
    scale = jnp.where(pl.program_id(1) == 1, 1.0 / S, 1.0)
    d = lax.dot_general(f, w, (((1,), (1,)), ((), ())),
                        preferred_element_type=jnp.float32)
    out_ref[...] = (d * scale).astype(jnp.bfloat16)


def kernel(nodes, neigh_idx, features, W):
    B = nodes.shape[0]
    N = features.shape[0]

    FBLK = 1000
    fgrid = N // FBLK + 1
    NT = fgrid * FBLK
    feat_p = jnp.pad(features, ((0, NT - N), (0, 0)))
    w3 = jnp.transpose(W.reshape(128, 2, D), (1, 0, 2))
    table = pl.pallas_call(
        _table_body,
        grid=(fgrid, 2),
        in_specs=[
            pl.BlockSpec((1, 128, D), lambda i, j: (j, 0, 0)),
            pl.BlockSpec((FBLK, D), lambda i, j: (i, 0)),
        ],
        out_specs=pl.BlockSpec((FBLK, D), lambda i, j: (j * fgrid + i, 0)),
        out_shape=jax.ShapeDtypeStruct((2 * NT, D), jnp.bfloat16),
    )(w3, feat_p)
    table = lax.bitcast_convert_type(table.reshape(2 * NT, DW, 2), jnp.int32)

    Bp = -(-B // (NW * 2 * CHUNK)) * (NW * 2 * CHUNK)
    npad = Bp - B
    pad_nodes = (jnp.arange(npad, dtype=nodes.dtype) * 1031) % N
    pad_neigh = ((jnp.arange(npad * S, dtype=nodes.dtype) * 523) % N
                 ).reshape(npad, S)
    nodes_p = jnp.concatenate([nodes, pad_nodes])
    nidx_p = jnp.concatenate([neigh_idx, pad_neigh], axis=0)
    dummy = N + (jnp.arange(Bp, dtype=nodes.dtype) % (NT - N))
    idx_all = jnp.concatenate([nodes_p[:, None], nidx_p + NT, dummy[:, None]],
                              axis=1).reshape(Bp * RPN)

    out_de = _make_sc(Bp)(idx_all, table)
    out_bt = jnp.stack([out_de[:, :DW], out_de[:, DW:]], axis=-1
                       ).reshape(Bp, D)
    return out_bt[:B].T


# speedup vs baseline: 2.5896x; 2.5896x over previous
import jax
import jax.numpy as jnp
from jax import lax
from jax.experimental import pallas as pl
from jax.experimental.pallas import tpu as pltpu
from jax.experimental.pallas import tpu_sc as plsc

D = 128
S = 10
RPN = S + 1
NC, NS, L = 2, 16, 16
NW = NC * NS
CHUNK = 32
CR = CHUNK * RPN
IPG = 88
NG = CR // IPG


def _sc_body(idx_hbm, table_hbm, out_hbm,
             idx_v, rows0, rows1, outc0, outc1,
             gsem0, gsem1, osem0, osem1, *, bpw, nchunks):
    wid = lax.axis_index("s") * NC + lax.axis_index("c")
    base = wid * bpw
    rows = (rows0, rows1)
    outc = (outc0, outc1)
    gsem = (gsem0, gsem1)
    osem = (osem0, osem1)

    pltpu.sync_copy(idx_hbm.at[pl.ds(base * RPN, bpw * RPN)], idx_v)

    def fire_gathers(b, c):
        for j in range(NG):
            pltpu.async_copy(
                table_hbm.at[idx_v.at[pl.ds(c * CR + j * IPG, IPG)]],
                rows[b].at[pl.ds(j * IPG, IPG)], gsem[b])

    def drain_gathers(b):
        for j in range(NG):
            pltpu.make_async_copy(
                table_hbm.at[idx_v.at[pl.ds(j * IPG, IPG)]],
                rows[b].at[pl.ds(j * IPG, IPG)], gsem[b]).wait()

    def compute(b):
        def node_body(i, carry):
            r0 = i * RPN
            for v in range(D // L):
                sl = pl.ds(v * L, L)
                acc = rows[b][r0, sl]
                for s in range(1, RPN):
                    acc = acc + rows[b][r0 + s, sl]
                outc[b][i, sl] = jnp.maximum(acc, 0.0)
            return carry
        lax.fori_loop(0, CHUNK, node_body, 0)

    def fire_out(b, c):
        pltpu.async_copy(outc[b], out_hbm.at[pl.ds(base + c * CHUNK, CHUNK)],
                         osem[b])

    def drain_out(b):
        pltpu.make_async_copy(outc[b], out_hbm.at[pl.ds(base, CHUNK)],
                              osem[b]).wait()

    fire_gathers(0, 0)

    def superstep(ss, carry):
        c0 = 2 * ss

        @pl.when(ss > 0)
        def _():
            drain_out(1)
        fire_gathers(1, c0 + 1)
        drain_gathers(0)

        @pl.when(ss > 0)
        def _():
            drain_out(0)
        compute(0)
        fire_out(0, c0)

        @pl.when(c0 + 2 < nchunks)
        def _():
            fire_gathers(0, c0 + 2)
        drain_gathers(1)
        compute(1)
        fire_out(1, c0 + 1)
        return carry

    lax.fori_loop(0, nchunks // 2, superstep, 0)
    drain_out(0)
    drain_out(1)


def _make_sc(Bp):
    bpw = Bp // NW
    nchunks = bpw // CHUNK
    assert nchunks % 2 == 0
    mesh = plsc.VectorSubcoreMesh(core_axis_name="c", subcore_axis_name="s")

    def body(idx_hbm, table_hbm, out_hbm, *scratch):
        _sc_body(idx_hbm, table_hbm, out_hbm, *scratch,
                 bpw=bpw, nchunks=nchunks)

    return pl.kernel(
        body,
        out_type=jax.ShapeDtypeStruct((Bp, D), jnp.float32),
        mesh=mesh,
        scratch_types=[
            pltpu.VMEM((bpw * RPN,), jnp.int32),
            pltpu.VMEM((CR, D), jnp.float32),
            pltpu.VMEM((CR, D), jnp.float32),
            pltpu.VMEM((CHUNK, D), jnp.float32),
            pltpu.VMEM((CHUNK, D), jnp.float32),
            pltpu.SemaphoreType.DMA,
            pltpu.SemaphoreType.DMA,
            pltpu.SemaphoreType.DMA,
            pltpu.SemaphoreType.DMA,
        ],
    )


def _table_body(w_ref, f_ref, out_ref):
    f = f_ref[...]
    w = w_ref[0]
    scale = jnp.where(pl.program_id(1) == 1, 1.0 / S, 1.0)
    d = lax.dot_general(f, w, (((1,), (1,)), ((), ())),
                        preferred_element_type=jnp.float32)
    out_ref[...] = d * scale


def kernel(nodes, neigh_idx, features, W):
    B = nodes.shape[0]
    N = features.shape[0]

    FBLK = 1000
    fgrid = N // FBLK
    w3 = jnp.transpose(W.reshape(128, 2, D), (1, 0, 2))
    table = pl.pallas_call(
        _table_body,
        grid=(fgrid, 2),
        in_specs=[
            pl.BlockSpec((1, 128, D), lambda i, j: (j, 0, 0)),
            pl.BlockSpec((FBLK, D), lambda i, j: (i, 0)),
        ],
        out_specs=pl.BlockSpec((FBLK, D), lambda i, j: (j * fgrid + i, 0)),
        out_shape=jax.ShapeDtypeStruct((2 * N, D), jnp.float32),
    )(w3, features)

    Bp = -(-B // (NW * 2 * CHUNK)) * (NW * 2 * CHUNK)
    npad = Bp - B
    pad_nodes = (jnp.arange(npad, dtype=nodes.dtype) * 1031) % N
    pad_neigh = ((jnp.arange(npad * S, dtype=nodes.dtype) * 523) % N
                 ).reshape(npad, S)
    nodes_p = jnp.concatenate([nodes, pad_nodes])
    nidx_p = jnp.concatenate([neigh_idx, pad_neigh], axis=0)
    idx_all = jnp.concatenate([nodes_p[:, None], nidx_p + N],
                              axis=1).reshape(Bp * RPN)

    out_bt = _make_sc(Bp)(idx_all, table)
    return out_bt[:B].T


# speedup vs baseline: 2.6086x; 1.0073x over previous
import jax
import jax.numpy as jnp
from jax import lax
from jax.experimental import pallas as pl
from jax.experimental.pallas import tpu as pltpu
from jax.experimental.pallas import tpu_sc as plsc

D = 128
S = 10
NC, NS, L = 2, 16, 16
NW = NC * NS
CHUNK = 32
NR = CHUNK * S
IPG = 80
NG = NR // IPG


def _sc_body(idxs_hbm, idxn_hbm, table_hbm, out_hbm,
             idxs_v, idxn_v, srows0, srows1, nrows0, nrows1, outc0, outc1,
             gsem0, gsem1, osem0, osem1, *, bpw, nchunks):
    wid = lax.axis_index("s") * NC + lax.axis_index("c")
    base = wid * bpw
    srows = (srows0, srows1)
    nrows = (nrows0, nrows1)
    outc = (outc0, outc1)
    gsem = (gsem0, gsem1)
    osem = (osem0, osem1)

    pltpu.sync_copy(idxs_hbm.at[pl.ds(base, bpw)], idxs_v)
    pltpu.sync_copy(idxn_hbm.at[pl.ds(base * S, bpw * S)], idxn_v)

    def fire_gathers(b, c):
        pltpu.async_copy(
            table_hbm.at[idxs_v.at[pl.ds(c * CHUNK, CHUNK)]],
            srows[b], gsem[b])
        for j in range(NG):
            pltpu.async_copy(
                table_hbm.at[idxn_v.at[pl.ds(c * NR + j * IPG, IPG)]],
                nrows[b].at[pl.ds(j * IPG, IPG)], gsem[b])

    def drain_gathers(b):
        pltpu.make_async_copy(
            table_hbm.at[idxs_v.at[pl.ds(0, CHUNK)]], srows[b],
            gsem[b]).wait()
        for j in range(NG):
            pltpu.make_async_copy(
                table_hbm.at[idxn_v.at[pl.ds(j * IPG, IPG)]],
                nrows[b].at[pl.ds(j * IPG, IPG)], gsem[b]).wait()

    def compute(b):
        def node_body(i, carry):
            r0 = i * S
            for v in range(D // L):
                sl = pl.ds(v * L, L)
                acc = srows[b][i, sl]
                for s in range(S):
                    acc = acc + nrows[b][r0 + s, sl]
                outc[b][i, sl] = jnp.maximum(acc, 0.0)
            return carry
        lax.fori_loop(0, CHUNK, node_body, 0)

    def fire_out(b, c):
        pltpu.async_copy(outc[b], out_hbm.at[pl.ds(base + c * CHUNK, CHUNK)],
                         osem[b])

    def drain_out(b):
        pltpu.make_async_copy(outc[b], out_hbm.at[pl.ds(base, CHUNK)],
                              osem[b]).wait()

    fire_gathers(0, 0)

    def superstep(ss, carry):
        c0 = 2 * ss

        @pl.when(ss > 0)
        def _():
            drain_out(1)
        fire_gathers(1, c0 + 1)
        drain_gathers(0)

        @pl.when(ss > 0)
        def _():
            drain_out(0)
        compute(0)
        fire_out(0, c0)

        @pl.when(c0 + 2 < nchunks)
        def _():
            fire_gathers(0, c0 + 2)
        drain_gathers(1)
        compute(1)
        fire_out(1, c0 + 1)
        return carry

    lax.fori_loop(0, nchunks // 2, superstep, 0)
    drain_out(0)
    drain_out(1)


def _make_sc(Bp):
    bpw = Bp // NW
    nchunks = bpw // CHUNK
    assert nchunks % 2 == 0
    mesh = plsc.VectorSubcoreMesh(core_axis_name="c", subcore_axis_name="s")

    def body(idxs_hbm, idxn_hbm, table_hbm, out_hbm, *scratch):
        _sc_body(idxs_hbm, idxn_hbm, table_hbm, out_hbm, *scratch,
                 bpw=bpw, nchunks=nchunks)

    return pl.kernel(
        body,
        out_type=jax.ShapeDtypeStruct((Bp, D), jnp.float32),
        mesh=mesh,
        scratch_types=[
            pltpu.VMEM((bpw,), jnp.int32),
            pltpu.VMEM((bpw * S,), jnp.int32),
            pltpu.VMEM((CHUNK, D), jnp.float32),
            pltpu.VMEM((CHUNK, D), jnp.float32),
            pltpu.VMEM((NR, D), jnp.float32),
            pltpu.VMEM((NR, D), jnp.float32),
            pltpu.VMEM((CHUNK, D), jnp.float32),
            pltpu.VMEM((CHUNK, D), jnp.float32),
            pltpu.SemaphoreType.DMA,
            pltpu.SemaphoreType.DMA,
            pltpu.SemaphoreType.DMA,
            pltpu.SemaphoreType.DMA,
        ],
    )


def _table_body(w_ref, f_ref, out_ref):
    f = f_ref[...].astype(jnp.bfloat16)
    w = w_ref[0].astype(jnp.bfloat16)
    scale = jnp.where(pl.program_id(1) == 1, 1.0 / S, 1.0)
    d = lax.dot_general(f, w, (((1,), (1,)), ((), ())),
                        preferred_element_type=jnp.float32)
    out_ref[...] = d * scale


def kernel(nodes, neigh_idx, features, W):
    B = nodes.shape[0]
    N = features.shape[0]

    FBLK = 1000
    fgrid = N // FBLK
    w3 = jnp.transpose(W.reshape(128, 2, D), (1, 0, 2))
    table = pl.pallas_call(
        _table_body,
        grid=(fgrid, 2),
        in_specs=[
            pl.BlockSpec((1, 128, D), lambda i, j: (j, 0, 0)),
            pl.BlockSpec((FBLK, D), lambda i, j: (i, 0)),
        ],
        out_specs=pl.BlockSpec((FBLK, D), lambda i, j: (j * fgrid + i, 0)),
        out_shape=jax.ShapeDtypeStruct((2 * N, D), jnp.float32),
    )(w3, features)

    Bp = -(-B // (NW * 2 * CHUNK)) * (NW * 2 * CHUNK)
    npad = Bp - B
    pad_nodes = (jnp.arange(npad, dtype=nodes.dtype) * 1031) % N
    pad_neigh = ((jnp.arange(npad * S, dtype=nodes.dtype) * 523) % N
                 ).reshape(npad, S)
    idx_self = jnp.concatenate([nodes, pad_nodes])
    idx_neigh = (jnp.concatenate([neigh_idx, pad_neigh], axis=0) + N
                 ).reshape(Bp * S)

    out_bt = _make_sc(Bp)(idx_self, idx_neigh, table)
    return out_bt[:B].T


# speedup vs baseline: 2.7662x; 1.0604x over previous
import jax
import jax.numpy as jnp
from jax import lax
from jax.experimental import pallas as pl
from jax.experimental.pallas import tpu as pltpu
from jax.experimental.pallas import tpu_sc as plsc

D = 128
S = 10
NC, NS, L = 2, 16, 16
NW = NC * NS
CHUNK = 32
NR = CHUNK * S
IPG = 80
NG = NR // IPG


def _sc_body(idxs_hbm, idxn_hbm, table_hbm, out_hbm,
             idxs_v, idxn_v, srows0, srows1, nrows0, nrows1, outc0, outc1,
             gsem0, gsem1, osem0, osem1, *, bpw, nchunks):
    wid = lax.axis_index("s") * NC + lax.axis_index("c")
    base = wid * bpw
    srows = (srows0, srows1)
    nrows = (nrows0, nrows1)
    outc = (outc0, outc1)
    gsem = (gsem0, gsem1)
    osem = (osem0, osem1)

    pltpu.sync_copy(idxs_hbm.at[pl.ds(base, bpw)], idxs_v)
    pltpu.sync_copy(idxn_hbm.at[pl.ds(base * S, bpw * S)], idxn_v)

    def fire_gathers(b, c):
        pltpu.async_copy(
            table_hbm.at[idxs_v.at[pl.ds(c * CHUNK, CHUNK)]],
            srows[b], gsem[b])
        for j in range(NG):
            pltpu.async_copy(
                table_hbm.at[idxn_v.at[pl.ds(c * NR + j * IPG, IPG)]],
                nrows[b].at[pl.ds(j * IPG, IPG)], gsem[b])

    def drain_gathers(b):
        pltpu.make_async_copy(
            table_hbm.at[idxs_v.at[pl.ds(0, CHUNK)]], srows[b],
            gsem[b]).wait()
        for j in range(NG):
            pltpu.make_async_copy(
                table_hbm.at[idxn_v.at[pl.ds(j * IPG, IPG)]],
                nrows[b].at[pl.ds(j * IPG, IPG)], gsem[b]).wait()

    def compute(b):
        def node_body(i, carry):
            r0 = i * S
            for v in range(D // L):
                sl = pl.ds(v * L, L)
                acc = srows[b][i, sl]
                for s in range(S):
                    acc = acc + nrows[b][r0 + s, sl]
                outc[b][i, sl] = jnp.maximum(acc, 0.0)
            return carry
        lax.fori_loop(0, CHUNK, node_body, 0)

    def fire_out(b, c):
        pltpu.async_copy(outc[b], out_hbm.at[pl.ds(base + c * CHUNK, CHUNK)],
                         osem[b])

    def drain_out(b):
        pltpu.make_async_copy(outc[b], out_hbm.at[pl.ds(base, CHUNK)],
                              osem[b]).wait()

    fire_gathers(0, 0)

    def superstep(ss, carry):
        c0 = 2 * ss

        @pl.when(ss > 0)
        def _():
            drain_out(1)
        fire_gathers(1, c0 + 1)
        drain_gathers(0)

        @pl.when(ss > 0)
        def _():
            drain_out(0)
        compute(0)
        fire_out(0, c0)

        @pl.when(c0 + 2 < nchunks)
        def _():
            fire_gathers(0, c0 + 2)
        drain_gathers(1)
        compute(1)
        fire_out(1, c0 + 1)
        return carry

    lax.fori_loop(0, nchunks // 2, superstep, 0)
    drain_out(0)
    drain_out(1)


def _make_sc(Bp):
    bpw = Bp // NW
    nchunks = bpw // CHUNK
    assert nchunks % 2 == 0
    mesh = plsc.VectorSubcoreMesh(core_axis_name="c", subcore_axis_name="s")

    def body(idxs_hbm, idxn_hbm, table_hbm, out_hbm, *scratch):
        _sc_body(idxs_hbm, idxn_hbm, table_hbm, out_hbm, *scratch,
                 bpw=bpw, nchunks=nchunks)

    return pl.kernel(
        body,
        out_type=jax.ShapeDtypeStruct((Bp, D), jnp.float32),
        mesh=mesh,
        scratch_types=[
            pltpu.VMEM((bpw,), jnp.int32),
            pltpu.VMEM((bpw * S,), jnp.int32),
            pltpu.VMEM((CHUNK, D), jnp.float32),
            pltpu.VMEM((CHUNK, D), jnp.float32),
            pltpu.VMEM((NR, D), jnp.float32),
            pltpu.VMEM((NR, D), jnp.float32),
            pltpu.VMEM((CHUNK, D), jnp.float32),
            pltpu.VMEM((CHUNK, D), jnp.float32),
            pltpu.SemaphoreType.DMA,
            pltpu.SemaphoreType.DMA,
            pltpu.SemaphoreType.DMA,
            pltpu.SemaphoreType.DMA,
        ],
    )


def _table_body(w_ref, f_ref, out_ref):
    f = f_ref[...].astype(jnp.bfloat16)
    w = w_ref[0].astype(jnp.bfloat16)
    scale = jnp.where(pl.program_id(1) == 1, 1.0 / S, 1.0)
    d = lax.dot_general(f, w, (((1,), (1,)), ((), ())),
                        preferred_element_type=jnp.float32)
    out_ref[...] = d * scale


def kernel(nodes, neigh_idx, features, W):
    B = nodes.shape[0]
    N = features.shape[0]

    FBLK = 2000
    fgrid = N // FBLK
    w3 = jnp.transpose(W.reshape(128, 2, D), (1, 0, 2))
    table = pl.pallas_call(
        _table_body,
        grid=(fgrid, 2),
        in_specs=[
            pl.BlockSpec((1, 128, D), lambda i, j: (j, 0, 0)),
            pl.BlockSpec((FBLK, D), lambda i, j: (i, 0)),
        ],
        out_specs=pl.BlockSpec((FBLK, D), lambda i, j: (j * fgrid + i, 0)),
        out_shape=jax.ShapeDtypeStruct((2 * N, D), jnp.float32),
    )(w3, features)

    Bp = -(-B // (NW * 2 * CHUNK)) * (NW * 2 * CHUNK)
    npad = Bp - B
    pad_nodes = (jnp.arange(npad, dtype=nodes.dtype) * 1031) % N
    pad_neigh = (jnp.arange(npad * S, dtype=nodes.dtype) * 523) % N
    idx_self = jnp.concatenate([nodes, pad_nodes])
    idx_neigh = jnp.concatenate([neigh_idx.reshape(B * S), pad_neigh]) + N

    out_bt = _make_sc(Bp)(idx_self, idx_neigh, table)
    return out_bt[:B].T
